# initial kernel scaffold (unmeasured)
import jax
import jax.numpy as jnp
from jax import lax
from jax.experimental import pallas as pl
from jax.experimental.pallas import tpu as pltpu

N_DEV = 16
B_LOC = 2
SQ = 256
SKV = 256
HQ_LOC = 4
DH = 64
D_MODEL = 512
HD_LOC = HQ_LOC * DH


def kernel(x, Wq, K_ext, V_ext, Wo):
    my = lax.axis_index("i")
    K_loc = lax.dynamic_slice_in_dim(K_ext, my * B_LOC, B_LOC, 0).transpose(0, 2, 1, 3)
    V_loc = lax.dynamic_slice_in_dim(V_ext, my * B_LOC, B_LOC, 0).transpose(0, 2, 1, 3)

    def body(x_ref, wq_ref, k_hbm, v_hbm, wo_ref, out_ref,
             wq_g, wo_g, ctx_s, k_buf, v_buf,
             wq_send, wq_recv, wo_send, wo_recv, kv_sem):
        my_pos = lax.axis_index("i")
        left = lax.rem(my_pos - 1 + N_DEV, N_DEV)
        right = lax.rem(my_pos + 1, N_DEV)

        barrier = pltpu.get_barrier_semaphore()
        pl.semaphore_signal(barrier, inc=1, device_id=(left,),
                            device_id_type=pl.DeviceIdType.MESH)
        pl.semaphore_signal(barrier, inc=1, device_id=(right,),
                            device_id_type=pl.DeviceIdType.MESH)
        pl.semaphore_wait(barrier, 2)

        qi = lax.broadcasted_iota(jnp.int32, (SQ, SKV), 0) // 64
        kj = lax.broadcasted_iota(jnp.int32, (SQ, SKV), 1) // 64
        mask = (qi == kj) | (kj == 0) | (lax.rem(qi + kj, 3) == 0)

        wq_g[0, :, :] = wq_ref[:, :]
        wo_g[0, :, :] = wo_ref[:, :]

        def fetch_kv(h):
            src = lax.rem(my_pos - h + N_DEV, N_DEV)
            g0 = src * HQ_LOC
            ck = pltpu.make_async_copy(
                k_hbm.at[:, pl.ds(g0, HQ_LOC)], k_buf, kv_sem.at[0])
            cv = pltpu.make_async_copy(
                v_hbm.at[:, pl.ds(g0, HQ_LOC)], v_buf, kv_sem.at[1])
            ck.start()
            cv.start()
            ck.wait()
            cv.wait()

        def compute(h, first):
            wq_h = wq_g[h]
            wo_h = wo_g[h]
            for b in range(B_LOC):
                q_all = jnp.dot(x_ref[b], wq_h,
                                preferred_element_type=jnp.float32)
                for g in range(HQ_LOC):
                    q = q_all[:, g * DH:(g + 1) * DH]
                    k = k_buf[b, g]
                    v = v_buf[b, g]
                    s = lax.dot_general(
                        q, k, (((1,), (1,)), ((), ())),
                        preferred_element_type=jnp.float32) * 0.125
                    s = jnp.where(mask, s, -1e9)
                    m = jnp.max(s, axis=1, keepdims=True)
                    e = jnp.exp(s - m)
                    w = e / jnp.sum(e, axis=1, keepdims=True)
                    ctx_s[:, g * DH:(g + 1) * DH] = jnp.dot(
                        w, v, preferred_element_type=jnp.float32)
                contrib = jnp.dot(ctx_s[:, :], wo_h,
                                  preferred_element_type=jnp.float32)
                if first:
                    out_ref[b, :, :] = contrib
                else:
                    out_ref[b, :, :] = out_ref[b, :, :] + contrib

        fetch_kv(0)
        compute(0, first=True)

        def hop(h, carry):
            rq = pltpu.make_async_remote_copy(
                src_ref=wq_g.at[h - 1], dst_ref=wq_g.at[h],
                send_sem=wq_send.at[h], recv_sem=wq_recv.at[h],
                device_id=(right,), device_id_type=pl.DeviceIdType.MESH)
            ro = pltpu.make_async_remote_copy(
                src_ref=wo_g.at[h - 1], dst_ref=wo_g.at[h],
                send_sem=wo_send.at[h], recv_sem=wo_recv.at[h],
                device_id=(right,), device_id_type=pl.DeviceIdType.MESH)
            rq.start()
            ro.start()
            rq.wait()
            ro.wait()
            fetch_kv(h)
            compute(h, first=False)
            return carry

        lax.fori_loop(1, N_DEV, hop, 0)

    return pl.pallas_call(
        body,
        out_shape=jax.ShapeDtypeStruct((B_LOC, SQ, D_MODEL), jnp.float32),
        in_specs=[
            pl.BlockSpec(memory_space=pltpu.VMEM),
            pl.BlockSpec(memory_space=pltpu.VMEM),
            pl.BlockSpec(memory_space=pltpu.ANY),
            pl.BlockSpec(memory_space=pltpu.ANY),
            pl.BlockSpec(memory_space=pltpu.VMEM),
        ],
        out_specs=pl.BlockSpec(memory_space=pltpu.VMEM),
        scratch_shapes=[
            pltpu.VMEM((N_DEV, D_MODEL, HD_LOC), jnp.float32),
            pltpu.VMEM((N_DEV, HD_LOC, D_MODEL), jnp.float32),
            pltpu.VMEM((SQ, HD_LOC), jnp.float32),
            pltpu.VMEM((B_LOC, HQ_LOC, SKV, DH), jnp.float32),
            pltpu.VMEM((B_LOC, HQ_LOC, SKV, DH), jnp.float32),
            pltpu.SemaphoreType.DMA((N_DEV,)),
            pltpu.SemaphoreType.DMA((N_DEV,)),
            pltpu.SemaphoreType.DMA((N_DEV,)),
            pltpu.SemaphoreType.DMA((N_DEV,)),
            pltpu.SemaphoreType.DMA((2,)),
        ],
        compiler_params=pltpu.CompilerParams(collective_id=0),
    )(x, Wq, K_loc, V_loc, Wo)


# baseline (device time: 282229 ns/iter reference)
import jax
import jax.numpy as jnp
from jax import lax
from jax.experimental import pallas as pl
from jax.experimental.pallas import tpu as pltpu

N_DEV = 16
B_LOC = 2
SQ = 256
SKV = 256
HQ_LOC = 4
DH = 64
D_MODEL = 512
HD_LOC = HQ_LOC * DH


def kernel(x, Wq, K_ext, V_ext, Wo):
    my = lax.axis_index("i")
    K_loc = lax.dynamic_slice_in_dim(K_ext, my * B_LOC, B_LOC, 0).transpose(0, 2, 1, 3)
    V_loc = lax.dynamic_slice_in_dim(V_ext, my * B_LOC, B_LOC, 0).transpose(0, 2, 1, 3)

    def body(x_ref, wq_ref, k_hbm, v_hbm, wo_ref, out_ref,
             wq_g, wo_g, ctx_s, k_buf, v_buf,
             wq_send, wq_recv, wo_send, wo_recv, kv_sem):
        my_pos = lax.axis_index("i")
        left = lax.rem(my_pos - 1 + N_DEV, N_DEV)
        right = lax.rem(my_pos + 1, N_DEV)

        barrier = pltpu.get_barrier_semaphore()
        pl.semaphore_signal(barrier, inc=1, device_id=(left,),
                            device_id_type=pl.DeviceIdType.MESH)
        pl.semaphore_signal(barrier, inc=1, device_id=(right,),
                            device_id_type=pl.DeviceIdType.MESH)
        pl.semaphore_wait(barrier, 2)

        qi = lax.broadcasted_iota(jnp.int32, (SQ, SKV), 0) // 64
        kj = lax.broadcasted_iota(jnp.int32, (SQ, SKV), 1) // 64
        mask = (qi == kj) | (kj == 0) | (lax.rem(qi + kj, 3) == 0)

        wq_g[0, :, :] = wq_ref[:, :]
        wo_g[0, :, :] = wo_ref[:, :]

        def fetch_kv(h):
            src = lax.rem(my_pos - h + N_DEV, N_DEV)
            g0 = src * HQ_LOC
            ck = pltpu.make_async_copy(
                k_hbm.at[:, pl.ds(g0, HQ_LOC)], k_buf, kv_sem.at[0])
            cv = pltpu.make_async_copy(
                v_hbm.at[:, pl.ds(g0, HQ_LOC)], v_buf, kv_sem.at[1])
            ck.start()
            cv.start()
            ck.wait()
            cv.wait()

        def compute(h, first):
            wq_h = wq_g[h]
            wo_h = wo_g[h]
            for b in range(B_LOC):
                q_all = jnp.dot(x_ref[b], wq_h,
                                preferred_element_type=jnp.float32)
                for g in range(HQ_LOC):
                    q = q_all[:, g * DH:(g + 1) * DH]
                    k = k_buf[b, g]
                    v = v_buf[b, g]
                    s = lax.dot_general(
                        q, k, (((1,), (1,)), ((), ())),
                        preferred_element_type=jnp.float32) * 0.125
                    s = jnp.where(mask, s, -1e9)
                    m = jnp.max(s, axis=1, keepdims=True)
                    e = jnp.exp(s - m)
                    w = e / jnp.sum(e, axis=1, keepdims=True)
                    ctx_s[:, g * DH:(g + 1) * DH] = jnp.dot(
                        w, v, preferred_element_type=jnp.float32)
                contrib = jnp.dot(ctx_s[:, :], wo_h,
                                  preferred_element_type=jnp.float32)
                if first:
                    out_ref[b, :, :] = contrib
                else:
                    out_ref[b, :, :] = out_ref[b, :, :] + contrib

        fetch_kv(0)
        compute(0, first=True)

        def hop(h, carry):
            rq = pltpu.make_async_remote_copy(
                src_ref=wq_g.at[h - 1], dst_ref=wq_g.at[h],
                send_sem=wq_send.at[h], recv_sem=wq_recv.at[h],
                device_id=(right,), device_id_type=pl.DeviceIdType.MESH)
            ro = pltpu.make_async_remote_copy(
                src_ref=wo_g.at[h - 1], dst_ref=wo_g.at[h],
                send_sem=wo_send.at[h], recv_sem=wo_recv.at[h],
                device_id=(right,), device_id_type=pl.DeviceIdType.MESH)
            rq.start()
            ro.start()
            rq.wait()
            ro.wait()
            fetch_kv(h)
            compute(h, first=False)
            return carry

        lax.fori_loop(1, N_DEV, hop, 0)

    return pl.pallas_call(
        body,
        out_shape=jax.ShapeDtypeStruct((B_LOC, SQ, D_MODEL), jnp.float32),
        in_specs=[
            pl.BlockSpec(memory_space=pltpu.VMEM),
            pl.BlockSpec(memory_space=pltpu.VMEM),
            pl.BlockSpec(memory_space=pl.ANY),
            pl.BlockSpec(memory_space=pl.ANY),
            pl.BlockSpec(memory_space=pltpu.VMEM),
        ],
        out_specs=pl.BlockSpec(memory_space=pltpu.VMEM),
        scratch_shapes=[
            pltpu.VMEM((N_DEV, D_MODEL, HD_LOC), jnp.float32),
            pltpu.VMEM((N_DEV, HD_LOC, D_MODEL), jnp.float32),
            pltpu.VMEM((SQ, HD_LOC), jnp.float32),
            pltpu.VMEM((B_LOC, HQ_LOC, SKV, DH), jnp.float32),
            pltpu.VMEM((B_LOC, HQ_LOC, SKV, DH), jnp.float32),
            pltpu.SemaphoreType.DMA((N_DEV,)),
            pltpu.SemaphoreType.DMA((N_DEV,)),
            pltpu.SemaphoreType.DMA((N_DEV,)),
            pltpu.SemaphoreType.DMA((N_DEV,)),
            pltpu.SemaphoreType.DMA((2,)),
        ],
        compiler_params=pltpu.CompilerParams(collective_id=0),
    )(x, Wq, K_loc, V_loc, Wo)


# device time: 145615 ns/iter; 1.9382x vs baseline; 1.9382x over previous
import jax
import jax.numpy as jnp
from jax import lax
from jax.experimental import pallas as pl
from jax.experimental.pallas import tpu as pltpu

N_DEV = 16
HR = 8
HL = 7
B_LOC = 2
SQ = 256
SKV = 256
HQ_LOC = 4
DH = 64
D_MODEL = 512
HD_LOC = HQ_LOC * DH


def kernel(x, Wq, K_ext, V_ext, Wo):
    my = lax.axis_index("i")
    K_loc = lax.dynamic_slice_in_dim(K_ext, my * B_LOC, B_LOC, 0).transpose(0, 2, 1, 3)
    V_loc = lax.dynamic_slice_in_dim(V_ext, my * B_LOC, B_LOC, 0).transpose(0, 2, 1, 3)

    def body(x_ref, wq_ref, k_hbm, v_hbm, wo_ref, out_ref,
             wqR, woR, wqL, woL, ctx_s, k_buf, v_buf,
             sqR, rqR, soR, roR, sqL, rqL, soL, roL, kv_sem):
        my_pos = lax.axis_index("i")
        left = lax.rem(my_pos - 1 + N_DEV, N_DEV)
        right = lax.rem(my_pos + 1, N_DEV)

        barrier = pltpu.get_barrier_semaphore()
        pl.semaphore_signal(barrier, inc=1, device_id=(left,),
                            device_id_type=pl.DeviceIdType.MESH)
        pl.semaphore_signal(barrier, inc=1, device_id=(right,),
                            device_id_type=pl.DeviceIdType.MESH)
        pl.semaphore_wait(barrier, 2)

        qi = lax.broadcasted_iota(jnp.int32, (SQ, SKV), 0) // 64
        kj = lax.broadcasted_iota(jnp.int32, (SQ, SKV), 1) // 64
        mask = (qi == kj) | (kj == 0) | (lax.rem(qi + kj, 3) == 0)

        wqR[0, :, :] = wq_ref[:, :]
        woR[0, :, :] = wo_ref[:, :]
        wqL[0, :, :] = wq_ref[:, :]
        woL[0, :, :] = wo_ref[:, :]

        def rdR(h):
            q = pltpu.make_async_remote_copy(
                src_ref=wqR.at[h - 1], dst_ref=wqR.at[h],
                send_sem=sqR.at[h], recv_sem=rqR.at[h],
                device_id=(right,), device_id_type=pl.DeviceIdType.MESH)
            o = pltpu.make_async_remote_copy(
                src_ref=woR.at[h - 1], dst_ref=woR.at[h],
                send_sem=soR.at[h], recv_sem=roR.at[h],
                device_id=(right,), device_id_type=pl.DeviceIdType.MESH)
            return q, o

        def rdL(h):
            q = pltpu.make_async_remote_copy(
                src_ref=wqL.at[h - 1], dst_ref=wqL.at[h],
                send_sem=sqL.at[h], recv_sem=rqL.at[h],
                device_id=(left,), device_id_type=pl.DeviceIdType.MESH)
            o = pltpu.make_async_remote_copy(
                src_ref=woL.at[h - 1], dst_ref=woL.at[h],
                send_sem=soL.at[h], recv_sem=roL.at[h],
                device_id=(left,), device_id_type=pl.DeviceIdType.MESH)
            return q, o

        def kv_descr(d, h):
            src = lax.rem(my_pos + (h if d else -h) + N_DEV, N_DEV)
            g0 = src * HQ_LOC
            kidx = d * 2 + lax.rem(h, 2)
            ck = pltpu.make_async_copy(
                k_hbm.at[:, pl.ds(g0, HQ_LOC)], k_buf.at[kidx], kv_sem.at[kidx, 0])
            cv = pltpu.make_async_copy(
                v_hbm.at[:, pl.ds(g0, HQ_LOC)], v_buf.at[kidx], kv_sem.at[kidx, 1])
            return ck, cv

        def start_fetch(d, h):
            ck, cv = kv_descr(d, h)
            ck.start()
            cv.start()

        def wait_fetch(d, h):
            ck, cv = kv_descr(d, h)
            ck.wait()
            cv.wait()

        def compute(wq_h, wo_h, kidx, first):
            k_blk = k_buf[kidx]
            v_blk = v_buf[kidx]
            for b in range(B_LOC):
                q_all = jnp.dot(x_ref[b], wq_h,
                                preferred_element_type=jnp.float32)
                for g in range(HQ_LOC):
                    q = q_all[:, g * DH:(g + 1) * DH]
                    k = k_blk[b, g]
                    v = v_blk[b, g]
                    s = lax.dot_general(
                        q, k, (((1,), (1,)), ((), ())),
                        preferred_element_type=jnp.float32) * 0.125
                    s = jnp.where(mask, s, -1e9)
                    m = jnp.max(s, axis=1, keepdims=True)
                    e = jnp.exp(s - m)
                    w = e / jnp.sum(e, axis=1, keepdims=True)
                    ctx_s[:, g * DH:(g + 1) * DH] = jnp.dot(
                        w, v, preferred_element_type=jnp.float32)
                contrib = jnp.dot(ctx_s[:, :], wo_h,
                                  preferred_element_type=jnp.float32)
                if first:
                    out_ref[b, :, :] = contrib
                else:
                    out_ref[b, :, :] = out_ref[b, :, :] + contrib

        q1, o1 = rdR(1)
        q1.start()
        o1.start()
        q2, o2 = rdL(1)
        q2.start()
        o2.start()
        start_fetch(0, 0)
        wait_fetch(0, 0)
        start_fetch(0, 1)
        start_fetch(1, 1)
        compute(wqR[0], woR[0], 0, first=True)

        def hop(h, carry):
            for d in rdR(h) + rdL(h):
                d.wait_recv()
            for d in rdR(h + 1):
                d.start()

            @pl.when(h < HL)
            def _():
                for d in rdL(h + 1):
                    d.start()

            wait_fetch(0, h)
            wait_fetch(1, h)
            start_fetch(0, h + 1)

            @pl.when(h < HL)
            def _():
                start_fetch(1, h + 1)

            compute(wqR[h], woR[h], lax.rem(h, 2), first=False)
            compute(wqL[h], woL[h], 2 + lax.rem(h, 2), first=False)
            for d in rdR(h) + rdL(h):
                d.wait_send()
            return carry

        lax.fori_loop(1, HR, hop, 0)

        q8, o8 = rdR(HR)
        q8.wait_recv()
        o8.wait_recv()
        wait_fetch(0, HR)
        compute(wqR[HR], woR[HR], lax.rem(HR, 2), first=False)
        q8.wait_send()
        o8.wait_send()

    return pl.pallas_call(
        body,
        out_shape=jax.ShapeDtypeStruct((B_LOC, SQ, D_MODEL), jnp.float32),
        in_specs=[
            pl.BlockSpec(memory_space=pltpu.VMEM),
            pl.BlockSpec(memory_space=pltpu.VMEM),
            pl.BlockSpec(memory_space=pl.ANY),
            pl.BlockSpec(memory_space=pl.ANY),
            pl.BlockSpec(memory_space=pltpu.VMEM),
        ],
        out_specs=pl.BlockSpec(memory_space=pltpu.VMEM),
        scratch_shapes=[
            pltpu.VMEM((HR + 1, D_MODEL, HD_LOC), jnp.float32),
            pltpu.VMEM((HR + 1, HD_LOC, D_MODEL), jnp.float32),
            pltpu.VMEM((HL + 1, D_MODEL, HD_LOC), jnp.float32),
            pltpu.VMEM((HL + 1, HD_LOC, D_MODEL), jnp.float32),
            pltpu.VMEM((SQ, HD_LOC), jnp.float32),
            pltpu.VMEM((4, B_LOC, HQ_LOC, SKV, DH), jnp.float32),
            pltpu.VMEM((4, B_LOC, HQ_LOC, SKV, DH), jnp.float32),
            pltpu.SemaphoreType.DMA((HR + 1,)),
            pltpu.SemaphoreType.DMA((HR + 1,)),
            pltpu.SemaphoreType.DMA((HR + 1,)),
            pltpu.SemaphoreType.DMA((HR + 1,)),
            pltpu.SemaphoreType.DMA((HL + 1,)),
            pltpu.SemaphoreType.DMA((HL + 1,)),
            pltpu.SemaphoreType.DMA((HL + 1,)),
            pltpu.SemaphoreType.DMA((HL + 1,)),
            pltpu.SemaphoreType.DMA((4, 2)),
        ],
        compiler_params=pltpu.CompilerParams(collective_id=0),
    )(x, Wq, K_loc, V_loc, Wo)


# device time: 140152 ns/iter; 2.0137x vs baseline; 1.0390x over previous
import jax
import jax.numpy as jnp
from jax import lax
from jax.experimental import pallas as pl
from jax.experimental.pallas import tpu as pltpu

N_DEV = 16
HR = 8
HL = 7
B_LOC = 2
SQ = 256
SKV = 256
HQ_LOC = 4
DH = 64
D_MODEL = 512
HD_LOC = HQ_LOC * DH


def kernel(x, Wq, K_ext, V_ext, Wo):
    my = lax.axis_index("i")
    WoT = Wo.T
    K_loc = lax.dynamic_slice_in_dim(K_ext, my * B_LOC, B_LOC, 0).transpose(0, 2, 1, 3)
    V_loc = lax.dynamic_slice_in_dim(V_ext, my * B_LOC, B_LOC, 0).transpose(0, 2, 1, 3)

    def body(x_ref, wq_ref, k_hbm, v_hbm, wot_ref, out_ref,
             gR, gL, ctx_s, k_buf, v_buf,
             sR, rR, sL, rL, kv_sem):
        my_pos = lax.axis_index("i")
        left = lax.rem(my_pos - 1 + N_DEV, N_DEV)
        right = lax.rem(my_pos + 1, N_DEV)

        barrier = pltpu.get_barrier_semaphore()
        pl.semaphore_signal(barrier, inc=1, device_id=(left,),
                            device_id_type=pl.DeviceIdType.MESH)
        pl.semaphore_signal(barrier, inc=1, device_id=(right,),
                            device_id_type=pl.DeviceIdType.MESH)
        pl.semaphore_wait(barrier, 2)

        qi = lax.broadcasted_iota(jnp.int32, (SQ, SKV), 0) // 64
        kj = lax.broadcasted_iota(jnp.int32, (SQ, SKV), 1) // 64
        mask = (qi == kj) | (kj == 0) | (lax.rem(qi + kj, 3) == 0)

        gR[0, 0, :, :] = wq_ref[:, :]
        gR[0, 1, :, :] = wot_ref[:, :]
        gL[0, 0, :, :] = wq_ref[:, :]
        gL[0, 1, :, :] = wot_ref[:, :]

        def rdR(h):
            return pltpu.make_async_remote_copy(
                src_ref=gR.at[h - 1], dst_ref=gR.at[h],
                send_sem=sR.at[h], recv_sem=rR.at[h],
                device_id=(right,), device_id_type=pl.DeviceIdType.MESH)

        def rdL(h):
            return pltpu.make_async_remote_copy(
                src_ref=gL.at[h - 1], dst_ref=gL.at[h],
                send_sem=sL.at[h], recv_sem=rL.at[h],
                device_id=(left,), device_id_type=pl.DeviceIdType.MESH)

        def kv_descr(d, h):
            src = lax.rem(my_pos + (h if d else -h) + N_DEV, N_DEV)
            g0 = src * HQ_LOC
            kidx = d * 2 + lax.rem(h, 2)
            ck = pltpu.make_async_copy(
                k_hbm.at[:, pl.ds(g0, HQ_LOC)], k_buf.at[kidx], kv_sem.at[kidx, 0])
            cv = pltpu.make_async_copy(
                v_hbm.at[:, pl.ds(g0, HQ_LOC)], v_buf.at[kidx], kv_sem.at[kidx, 1])
            return ck, cv

        def start_fetch(d, h):
            ck, cv = kv_descr(d, h)
            ck.start()
            cv.start()

        def wait_fetch(d, h):
            ck, cv = kv_descr(d, h)
            ck.wait()
            cv.wait()

        def compute(g_ref, h, kidx, first):
            wq_h = g_ref[h, 0]
            wot_h = g_ref[h, 1]
            k_blk = k_buf[kidx]
            v_blk = v_buf[kidx]
            for b in range(B_LOC):
                q_all = jnp.dot(x_ref[b], wq_h,
                                preferred_element_type=jnp.float32)
                for g in range(HQ_LOC):
                    q = q_all[:, g * DH:(g + 1) * DH]
                    k = k_blk[b, g]
                    v = v_blk[b, g]
                    s = lax.dot_general(
                        q, k, (((1,), (1,)), ((), ())),
                        preferred_element_type=jnp.float32) * 0.125
                    s = jnp.where(mask, s, -1e9)
                    m = jnp.max(s, axis=1, keepdims=True)
                    e = jnp.exp(s - m)
                    w = e / jnp.sum(e, axis=1, keepdims=True)
                    ctx_s[:, g * DH:(g + 1) * DH] = jnp.dot(
                        w, v, preferred_element_type=jnp.float32)
                contrib = lax.dot_general(
                    ctx_s[:, :], wot_h, (((1,), (1,)), ((), ())),
                    preferred_element_type=jnp.float32)
                if first:
                    out_ref[b, :, :] = contrib
                else:
                    out_ref[b, :, :] = out_ref[b, :, :] + contrib

        rdR(1).start()
        rdL(1).start()
        start_fetch(0, 0)
        wait_fetch(0, 0)
        start_fetch(0, 1)
        start_fetch(1, 1)
        compute(gR, 0, 0, first=True)

        def hop(h, carry):
            rdR(h).wait_recv()
            rdR(h + 1).start()
            rdL(h).wait_recv()

            @pl.when(h < HL)
            def _():
                rdL(h + 1).start()

            wait_fetch(0, h)
            wait_fetch(1, h)
            start_fetch(0, h + 1)

            @pl.when(h < HL)
            def _():
                start_fetch(1, h + 1)

            compute(gR, h, lax.rem(h, 2), first=False)
            compute(gL, h, 2 + lax.rem(h, 2), first=False)
            rdR(h).wait_send()
            rdL(h).wait_send()
            return carry

        lax.fori_loop(1, HR, hop, 0)

        r8 = rdR(HR)
        r8.wait_recv()
        wait_fetch(0, HR)
        compute(gR, HR, lax.rem(HR, 2), first=False)
        r8.wait_send()

    return pl.pallas_call(
        body,
        out_shape=jax.ShapeDtypeStruct((B_LOC, SQ, D_MODEL), jnp.float32),
        in_specs=[
            pl.BlockSpec(memory_space=pltpu.VMEM),
            pl.BlockSpec(memory_space=pltpu.VMEM),
            pl.BlockSpec(memory_space=pl.ANY),
            pl.BlockSpec(memory_space=pl.ANY),
            pl.BlockSpec(memory_space=pltpu.VMEM),
        ],
        out_specs=pl.BlockSpec(memory_space=pltpu.VMEM),
        scratch_shapes=[
            pltpu.VMEM((HR + 1, 2, D_MODEL, HD_LOC), jnp.float32),
            pltpu.VMEM((HL + 1, 2, D_MODEL, HD_LOC), jnp.float32),
            pltpu.VMEM((SQ, HD_LOC), jnp.float32),
            pltpu.VMEM((4, B_LOC, HQ_LOC, SKV, DH), jnp.float32),
            pltpu.VMEM((4, B_LOC, HQ_LOC, SKV, DH), jnp.float32),
            pltpu.SemaphoreType.DMA((HR + 1,)),
            pltpu.SemaphoreType.DMA((HR + 1,)),
            pltpu.SemaphoreType.DMA((HL + 1,)),
            pltpu.SemaphoreType.DMA((HL + 1,)),
            pltpu.SemaphoreType.DMA((4, 2)),
        ],
        compiler_params=pltpu.CompilerParams(collective_id=0),
    )(x, Wq, K_loc, V_loc, WoT)


# device time: 139886 ns/iter; 2.0176x vs baseline; 1.0019x over previous
import jax
import jax.numpy as jnp
from jax import lax
from jax.experimental import pallas as pl
from jax.experimental.pallas import tpu as pltpu

N_DEV = 16
HR = 8
HL = 7
B_LOC = 2
SQ = 256
SKV = 256
HQ_LOC = 4
DH = 64
D_MODEL = 512
HD_LOC = HQ_LOC * DH


def kernel(x, Wq, K_ext, V_ext, Wo):
    my = lax.axis_index("i")
    WoT = Wo.T
    x2 = x.reshape(B_LOC * SQ, D_MODEL)
    K_loc = lax.dynamic_slice_in_dim(K_ext, my * B_LOC, B_LOC, 0).transpose(0, 2, 1, 3)
    V_loc = lax.dynamic_slice_in_dim(V_ext, my * B_LOC, B_LOC, 0).transpose(0, 2, 1, 3)

    def body(x_ref, wq_ref, k_hbm, v_hbm, wot_ref, out_ref,
             gR, gL, ctx_s, k_buf, v_buf,
             sR, rR, sL, rL, kv_sem):
        my_pos = lax.axis_index("i")
        left = lax.rem(my_pos - 1 + N_DEV, N_DEV)
        right = lax.rem(my_pos + 1, N_DEV)

        barrier = pltpu.get_barrier_semaphore()
        pl.semaphore_signal(barrier, inc=1, device_id=(left,),
                            device_id_type=pl.DeviceIdType.MESH)
        pl.semaphore_signal(barrier, inc=1, device_id=(right,),
                            device_id_type=pl.DeviceIdType.MESH)
        pl.semaphore_wait(barrier, 2)

        qi = lax.broadcasted_iota(jnp.int32, (SQ, SKV), 0) // 64
        kj = lax.broadcasted_iota(jnp.int32, (SQ, SKV), 1) // 64
        mask = (qi == kj) | (kj == 0) | (lax.rem(qi + kj, 3) == 0)
        bias = jnp.where(mask, 0.0, -30.0).astype(jnp.float32)

        gR[0, 0, :, :] = wq_ref[:, :]
        gR[0, 1, :, :] = wot_ref[:, :]
        gL[0, 0, :, :] = wq_ref[:, :]
        gL[0, 1, :, :] = wot_ref[:, :]

        def rdR(h):
            return pltpu.make_async_remote_copy(
                src_ref=gR.at[h - 1], dst_ref=gR.at[h],
                send_sem=sR.at[h], recv_sem=rR.at[h],
                device_id=(right,), device_id_type=pl.DeviceIdType.MESH)

        def rdL(h):
            return pltpu.make_async_remote_copy(
                src_ref=gL.at[h - 1], dst_ref=gL.at[h],
                send_sem=sL.at[h], recv_sem=rL.at[h],
                device_id=(left,), device_id_type=pl.DeviceIdType.MESH)

        def kv_descr(d, h):
            src = lax.rem(my_pos + (h if d else -h) + N_DEV, N_DEV)
            g0 = src * HQ_LOC
            kidx = d * 2 + lax.rem(h, 2)
            ck = pltpu.make_async_copy(
                k_hbm.at[:, pl.ds(g0, HQ_LOC)], k_buf.at[kidx], kv_sem.at[kidx, 0])
            cv = pltpu.make_async_copy(
                v_hbm.at[:, pl.ds(g0, HQ_LOC)], v_buf.at[kidx], kv_sem.at[kidx, 1])
            return ck, cv

        def start_fetch(d, h):
            ck, cv = kv_descr(d, h)
            ck.start()
            cv.start()

        def wait_fetch(d, h):
            ck, cv = kv_descr(d, h)
            ck.wait()
            cv.wait()

        def compute(g_ref, h, kidx, first):
            wq_h = g_ref[h, 0]
            wot_h = g_ref[h, 1]
            k_blk = k_buf[kidx]
            v_blk = v_buf[kidx]
            q_2b = jnp.dot(x_ref[:, :], wq_h,
                           preferred_element_type=jnp.float32)
            for b in range(B_LOC):
                q_all = q_2b[b * SQ:(b + 1) * SQ, :]
                for g in range(HQ_LOC):
                    q = q_all[:, g * DH:(g + 1) * DH]
                    k = k_blk[b, g]
                    v = v_blk[b, g]
                    s = lax.dot_general(
                        q, k, (((1,), (1,)), ((), ())),
                        preferred_element_type=jnp.float32)
                    e = jnp.exp(s * 0.125 + bias)
                    recip = 1.0 / jnp.sum(e, axis=1, keepdims=True)
                    ctx_s[:, g * DH:(g + 1) * DH] = jnp.dot(
                        e, v, preferred_element_type=jnp.float32) * recip
                contrib = lax.dot_general(
                    ctx_s[:, :], wot_h, (((1,), (1,)), ((), ())),
                    preferred_element_type=jnp.float32)
                if first:
                    out_ref[b, :, :] = contrib
                else:
                    out_ref[b, :, :] = out_ref[b, :, :] + contrib

        rdR(1).start()
        rdL(1).start()
        start_fetch(0, 0)
        wait_fetch(0, 0)
        start_fetch(0, 1)
        start_fetch(1, 1)
        compute(gR, 0, 0, first=True)

        def hop(h, carry):
            rdR(h).wait_recv()
            rdR(h + 1).start()
            rdL(h).wait_recv()

            @pl.when(h < HL)
            def _():
                rdL(h + 1).start()

            wait_fetch(0, h)
            wait_fetch(1, h)
            start_fetch(0, h + 1)

            @pl.when(h < HL)
            def _():
                start_fetch(1, h + 1)

            compute(gR, h, lax.rem(h, 2), first=False)
            compute(gL, h, 2 + lax.rem(h, 2), first=False)
            rdR(h).wait_send()
            rdL(h).wait_send()
            return carry

        lax.fori_loop(1, HR, hop, 0)

        r8 = rdR(HR)
        r8.wait_recv()
        wait_fetch(0, HR)
        compute(gR, HR, lax.rem(HR, 2), first=False)
        r8.wait_send()

    return pl.pallas_call(
        body,
        out_shape=jax.ShapeDtypeStruct((B_LOC, SQ, D_MODEL), jnp.float32),
        in_specs=[
            pl.BlockSpec(memory_space=pltpu.VMEM),
            pl.BlockSpec(memory_space=pltpu.VMEM),
            pl.BlockSpec(memory_space=pl.ANY),
            pl.BlockSpec(memory_space=pl.ANY),
            pl.BlockSpec(memory_space=pltpu.VMEM),
        ],
        out_specs=pl.BlockSpec(memory_space=pltpu.VMEM),
        scratch_shapes=[
            pltpu.VMEM((HR + 1, 2, D_MODEL, HD_LOC), jnp.float32),
            pltpu.VMEM((HL + 1, 2, D_MODEL, HD_LOC), jnp.float32),
            pltpu.VMEM((SQ, HD_LOC), jnp.float32),
            pltpu.VMEM((4, B_LOC, HQ_LOC, SKV, DH), jnp.float32),
            pltpu.VMEM((4, B_LOC, HQ_LOC, SKV, DH), jnp.float32),
            pltpu.SemaphoreType.DMA((HR + 1,)),
            pltpu.SemaphoreType.DMA((HR + 1,)),
            pltpu.SemaphoreType.DMA((HL + 1,)),
            pltpu.SemaphoreType.DMA((HL + 1,)),
            pltpu.SemaphoreType.DMA((4, 2)),
        ],
        compiler_params=pltpu.CompilerParams(collective_id=0),
    )(x2, Wq, K_loc, V_loc, WoT)


# device time: 129098 ns/iter; 2.1862x vs baseline; 1.0836x over previous
import jax
import jax.numpy as jnp
from jax import lax
from jax.experimental import pallas as pl
from jax.experimental.pallas import tpu as pltpu

N_DEV = 16
HR = 8
HL = 7
B_LOC = 2
SQ = 256
SKV = 256
HQ_LOC = 4
DH = 64
D_MODEL = 512
HD_LOC = HQ_LOC * DH


def kernel(x, Wq, K_ext, V_ext, Wo):
    my = lax.axis_index("i")
    WoT = Wo.T
    x2 = x.reshape(B_LOC * SQ, D_MODEL)
    K_loc = lax.dynamic_slice_in_dim(K_ext, my * B_LOC, B_LOC, 0).transpose(0, 2, 1, 3)
    V_loc = lax.dynamic_slice_in_dim(V_ext, my * B_LOC, B_LOC, 0).transpose(0, 2, 1, 3)

    def body(x_ref, wq_ref, k_hbm, v_hbm, wot_ref, out_ref,
             gR, gL, ctx_s, k_buf, v_buf,
             sR, rR, sL, rL, kv_sem):
        my_pos = lax.axis_index("i")
        left = lax.rem(my_pos - 1 + N_DEV, N_DEV)
        right = lax.rem(my_pos + 1, N_DEV)

        barrier = pltpu.get_barrier_semaphore()
        pl.semaphore_signal(barrier, inc=1, device_id=(left,),
                            device_id_type=pl.DeviceIdType.MESH)
        pl.semaphore_signal(barrier, inc=1, device_id=(right,),
                            device_id_type=pl.DeviceIdType.MESH)
        pl.semaphore_wait(barrier, 2)

        qi = lax.broadcasted_iota(jnp.int32, (SQ, SKV), 0) // 64
        kj = lax.broadcasted_iota(jnp.int32, (SQ, SKV), 1) // 64
        mask = (qi == kj) | (kj == 0) | (lax.rem(qi + kj, 3) == 0)
        bias = jnp.where(mask, 0.0, -30.0).astype(jnp.float32)

        gR[0, 0, :, :] = wq_ref[:, :]
        gR[0, 1, :, :] = wot_ref[:, :]
        gL[0, 0, :, :] = wq_ref[:, :]
        gL[0, 1, :, :] = wot_ref[:, :]

        def rdR(h, s):
            return pltpu.make_async_remote_copy(
                src_ref=gR.at[h - 1, s], dst_ref=gR.at[h, s],
                send_sem=sR.at[h, s], recv_sem=rR.at[h, s],
                device_id=(right,), device_id_type=pl.DeviceIdType.MESH)

        def rdL(h, s):
            return pltpu.make_async_remote_copy(
                src_ref=gL.at[h - 1, s], dst_ref=gL.at[h, s],
                send_sem=sL.at[h, s], recv_sem=rL.at[h, s],
                device_id=(left,), device_id_type=pl.DeviceIdType.MESH)

        def kv_descr(d, h):
            src = lax.rem(my_pos + (h if d else -h) + N_DEV, N_DEV)
            g0 = src * HQ_LOC
            kidx = d * 2 + lax.rem(h, 2)
            ck = pltpu.make_async_copy(
                k_hbm.at[:, pl.ds(g0, HQ_LOC)], k_buf.at[kidx], kv_sem.at[kidx, 0])
            cv = pltpu.make_async_copy(
                v_hbm.at[:, pl.ds(g0, HQ_LOC)], v_buf.at[kidx], kv_sem.at[kidx, 1])
            return ck, cv

        def start_fetch(d, h):
            ck, cv = kv_descr(d, h)
            ck.start()
            cv.start()

        def wait_fetch(d, h):
            ck, cv = kv_descr(d, h)
            ck.wait()
            cv.wait()

        def compute(g_ref, h, kidx, first):
            wq_h = g_ref[h, 0]
            wot_h = g_ref[h, 1]
            k_blk = k_buf[kidx]
            v_blk = v_buf[kidx]
            q_2b = jnp.dot(x_ref[:, :], wq_h,
                           preferred_element_type=jnp.float32)
            for b in range(B_LOC):
                q_all = q_2b[b * SQ:(b + 1) * SQ, :]
                for g in range(HQ_LOC):
                    q = q_all[:, g * DH:(g + 1) * DH]
                    k = k_blk[b, g]
                    v = v_blk[b, g]
                    s = lax.dot_general(
                        q, k, (((1,), (1,)), ((), ())),
                        preferred_element_type=jnp.float32)
                    e = jnp.exp(s * 0.125 + bias)
                    recip = 1.0 / jnp.sum(e, axis=1, keepdims=True)
                    ctx_s[:, g * DH:(g + 1) * DH] = jnp.dot(
                        e, v, preferred_element_type=jnp.float32) * recip
                contrib = lax.dot_general(
                    ctx_s[:, :], wot_h, (((1,), (1,)), ((), ())),
                    preferred_element_type=jnp.float32)
                if first:
                    out_ref[b, :, :] = contrib
                else:
                    out_ref[b, :, :] = out_ref[b, :, :] + contrib

        rdR(1, 0).start()
        rdL(1, 0).start()
        rdR(1, 1).start()
        rdL(1, 1).start()
        start_fetch(0, 0)
        wait_fetch(0, 0)
        start_fetch(0, 1)
        start_fetch(1, 1)
        compute(gR, 0, 0, first=True)

        def hop(h, carry):
            rdR(h, 0).wait_recv()
            rdR(h + 1, 0).start()
            rdL(h, 0).wait_recv()

            @pl.when(h < HL)
            def _():
                rdL(h + 1, 0).start()

            rdR(h, 1).wait_recv()
            rdR(h + 1, 1).start()
            rdL(h, 1).wait_recv()

            @pl.when(h < HL)
            def _():
                rdL(h + 1, 1).start()

            wait_fetch(0, h)
            wait_fetch(1, h)
            start_fetch(0, h + 1)

            @pl.when(h < HL)
            def _():
                start_fetch(1, h + 1)

            compute(gR, h, lax.rem(h, 2), first=False)
            compute(gL, h, 2 + lax.rem(h, 2), first=False)
            for s in range(2):
                rdR(h, s).wait_send()
                rdL(h, s).wait_send()
            return carry

        lax.fori_loop(1, HR, hop, 0)

        rdR(HR, 0).wait_recv()
        rdR(HR, 1).wait_recv()
        wait_fetch(0, HR)
        compute(gR, HR, lax.rem(HR, 2), first=False)
        rdR(HR, 0).wait_send()
        rdR(HR, 1).wait_send()

    return pl.pallas_call(
        body,
        out_shape=jax.ShapeDtypeStruct((B_LOC, SQ, D_MODEL), jnp.float32),
        in_specs=[
            pl.BlockSpec(memory_space=pltpu.VMEM),
            pl.BlockSpec(memory_space=pltpu.VMEM),
            pl.BlockSpec(memory_space=pl.ANY),
            pl.BlockSpec(memory_space=pl.ANY),
            pl.BlockSpec(memory_space=pltpu.VMEM),
        ],
        out_specs=pl.BlockSpec(memory_space=pltpu.VMEM),
        scratch_shapes=[
            pltpu.VMEM((HR + 1, 2, D_MODEL, HD_LOC), jnp.float32),
            pltpu.VMEM((HL + 1, 2, D_MODEL, HD_LOC), jnp.float32),
            pltpu.VMEM((SQ, HD_LOC), jnp.float32),
            pltpu.VMEM((4, B_LOC, HQ_LOC, SKV, DH), jnp.float32),
            pltpu.VMEM((4, B_LOC, HQ_LOC, SKV, DH), jnp.float32),
            pltpu.SemaphoreType.DMA((HR + 1, 2)),
            pltpu.SemaphoreType.DMA((HR + 1, 2)),
            pltpu.SemaphoreType.DMA((HL + 1, 2)),
            pltpu.SemaphoreType.DMA((HL + 1, 2)),
            pltpu.SemaphoreType.DMA((4, 2)),
        ],
        compiler_params=pltpu.CompilerParams(collective_id=0),
    )(x2, Wq, K_loc, V_loc, WoT)
